# baseline (speedup 1.0000x reference)
import jax
import jax.numpy as jnp
from jax import lax
from jax.experimental import pallas as pl
from jax.experimental.pallas import tpu as pltpu
from jax.experimental.pallas import tpu_sc as plsc

N_ATOMS = 16384
N_SYS = 16
D = 512
N_SPECIES = 100

NTILES = 16
CHUNK = 16
SPECIES_PAD = NTILES * CHUNK
APT = N_ATOMS // NTILES
NCHUNK = APT // CHUNK

_MAGIC = 0x5F3759DF


def _sc_body(an_hbm, px_hbm, py_hbm, pz_hbm, batch_hbm, epad_hbm, w_hbm,
             bvec_hbm, out_hbm,
             e_loc, w_loc, vchunk, v_loc, an_loc, seg_loc,
             px_loc, py_loc, pz_loc,
             acc_loc, red_loc, bvec_loc, tot_loc, shared_v, shared_acc, sem):
    wid = lax.axis_index("s")

    sl = pl.ds(wid * APT, APT)
    cp_an = pltpu.async_copy(an_hbm.at[sl], an_loc, sem)
    cp_seg = pltpu.async_copy(batch_hbm.at[sl], seg_loc, sem)
    cp_px = pltpu.async_copy(px_hbm.at[sl], px_loc, sem)
    cp_py = pltpu.async_copy(py_hbm.at[sl], py_loc, sem)
    cp_pz = pltpu.async_copy(pz_hbm.at[sl], pz_loc, sem)

    pltpu.sync_copy(epad_hbm.at[wid], e_loc)
    pltpu.sync_copy(w_hbm, w_loc)
    vreg = jnp.zeros((CHUNK,), jnp.float32)
    for d0 in range(0, D, CHUNK):
        wv = w_loc[pl.ds(d0, CHUNK)]
        for j in range(CHUNK):
            vreg = vreg + wv[j] * e_loc[d0 + j]
    vchunk[...] = vreg
    pltpu.sync_copy(vchunk, shared_v.at[pl.ds(wid * CHUNK, CHUNK)])
    plsc.subcore_barrier()
    pltpu.sync_copy(shared_v, v_loc)

    cp_an.wait()
    cp_seg.wait()
    cp_px.wait()
    cp_py.wait()
    cp_pz.wait()

    acc_loc[...] = jnp.zeros((N_SYS,), jnp.float32)

    def chunk_body(c, carry):
        base = c * CHUNK
        a = an_loc[pl.ds(base, CHUNK)]
        seg = seg_loc[pl.ds(base, CHUNK)]
        x = px_loc[pl.ds(base, CHUNK)]
        y = py_loc[pl.ds(base, CHUNK)]
        z = pz_loc[pl.ds(base, CHUNK)]
        rr = x * x + y * y + z * z
        w = plsc.bitcast(_MAGIC - (plsc.bitcast(rr, jnp.int32) >> 1), jnp.float32)
        half = rr * 0.5
        for _ in range(2):
            w = w * (1.5 - half * w * w)
        r = rr * w
        va = plsc.load_gather(v_loc, [a])
        plsc.addupdate_scatter(acc_loc, [seg], r * va)

    for c in range(NCHUNK):
        chunk_body(c, 0)

    pltpu.sync_copy(acc_loc, shared_acc.at[pl.ds(wid * N_SYS, N_SYS)])
    plsc.subcore_barrier()

    @pl.when(wid == 0)
    def _():
        pltpu.sync_copy(shared_acc, red_loc)
        pltpu.sync_copy(bvec_hbm, bvec_loc)
        tot = bvec_loc[...]
        for i in range(NTILES):
            tot = tot + red_loc[pl.ds(i * N_SYS, N_SYS)]
        tot_loc[...] = tot
        pltpu.sync_copy(tot_loc, out_hbm)


_sc_kernel = pl.kernel(
    _sc_body,
    out_type=jax.ShapeDtypeStruct((N_SYS,), jnp.float32),
    mesh=plsc.VectorSubcoreMesh(core_axis_name="c", subcore_axis_name="s",
                                num_cores=1, num_subcores=NTILES),
    compiler_params=pltpu.CompilerParams(needs_layout_passes=False),
    scratch_types=[
        pltpu.VMEM((D, CHUNK), jnp.float32),
        pltpu.VMEM((D,), jnp.float32),
        pltpu.VMEM((CHUNK,), jnp.float32),
        pltpu.VMEM((SPECIES_PAD,), jnp.float32),
        pltpu.VMEM((APT,), jnp.int32),
        pltpu.VMEM((APT,), jnp.int32),
        pltpu.VMEM((APT,), jnp.float32),
        pltpu.VMEM((APT,), jnp.float32),
        pltpu.VMEM((APT,), jnp.float32),
        pltpu.VMEM((N_SYS,), jnp.float32),
        pltpu.VMEM((NTILES * N_SYS,), jnp.float32),
        pltpu.VMEM((N_SYS,), jnp.float32),
        pltpu.VMEM((N_SYS,), jnp.float32),
        pltpu.VMEM_SHARED((SPECIES_PAD,), jnp.float32),
        pltpu.VMEM_SHARED((NTILES * N_SYS,), jnp.float32),
        pltpu.SemaphoreType.DMA,
    ],
)


def kernel(atomic_numbers, pos, batch, species_embed, W, b):
    an = atomic_numbers.astype(jnp.int32)
    bt = batch.astype(jnp.int32)
    posT = pos.astype(jnp.float32).T
    px, py, pz = posT[0], posT[1], posT[2]
    epad = jnp.pad(species_embed.astype(jnp.float32),
                   ((0, SPECIES_PAD - N_SPECIES), (0, 0)))
    epad = epad.reshape(NTILES, CHUNK, D).transpose(0, 2, 1)
    wf = W.reshape(-1).astype(jnp.float32)
    bvec = jnp.broadcast_to(b.astype(jnp.float32), (N_SYS,))
    y = _sc_kernel(an, px, py, pz, bt, epad, wf, bvec)
    return y.reshape(N_SYS, 1)


# speedup vs baseline: 1.0543x; 1.0543x over previous
import jax
import jax.numpy as jnp
from jax import lax
from jax.experimental import pallas as pl
from jax.experimental.pallas import tpu as pltpu
from jax.experimental.pallas import tpu_sc as plsc

N_ATOMS = 16384
N_SYS = 16
D = 512
N_SPECIES = 100

NTILES = 16
CHUNK = 16
NGROUP = 7
SPECIES_PAD = NGROUP * CHUNK
DHALF = D // 2
APT = N_ATOMS // NTILES
NCHUNK = APT // CHUNK

_MAGIC = 0x5F3759DF


def _sc_body(pk_hbm, e_hbm, w_hbm, bvec_hbm, out_hbm,
             pk_loc, e_loc, w_loc, vchunk, v0_loc, v1_loc, v_loc,
             acc_loc, red_loc, bvec_loc, tot_loc,
             shared_v0, shared_v1, shared_acc, sem):
    wid = lax.axis_index("s")

    cp_pk = pltpu.async_copy(pk_hbm.at[:, pl.ds(wid * APT, APT)], pk_loc, sem)

    @pl.when(wid < 2 * NGROUP)
    def _():
        g = wid // 2
        h = wid % 2
        pltpu.sync_copy(e_hbm.at[wid], e_loc)
        pltpu.sync_copy(w_hbm.at[pl.ds(h * DHALF, DHALF)], w_loc)
        vreg = jnp.zeros((CHUNK,), jnp.float32)
        for d0 in range(0, DHALF, CHUNK):
            wv = w_loc[pl.ds(d0, CHUNK)]
            for j in range(CHUNK):
                vreg = vreg + wv[j] * e_loc[d0 + j]
        vchunk[...] = vreg

        @pl.when(h == 0)
        def _():
            pltpu.sync_copy(vchunk, shared_v0.at[pl.ds(g * CHUNK, CHUNK)])

        @pl.when(h == 1)
        def _():
            pltpu.sync_copy(vchunk, shared_v1.at[pl.ds(g * CHUNK, CHUNK)])

    plsc.subcore_barrier()
    pltpu.sync_copy(shared_v0, v0_loc)
    pltpu.sync_copy(shared_v1, v1_loc)
    for c in range(NGROUP):
        sl = pl.ds(c * CHUNK, CHUNK)
        v_loc[sl] = v0_loc[sl] + v1_loc[sl]

    cp_pk.wait()

    acc_loc[...] = jnp.zeros((N_SYS,), jnp.float32)

    def chunk_body(c, carry):
        base = c * CHUNK
        sl = pl.ds(base, CHUNK)
        x = pk_loc[0, sl]
        y = pk_loc[1, sl]
        z = pk_loc[2, sl]
        a = plsc.bitcast(pk_loc[3, sl], jnp.int32)
        seg = plsc.bitcast(pk_loc[4, sl], jnp.int32)
        rr = x * x + y * y + z * z
        w = plsc.bitcast(_MAGIC - (plsc.bitcast(rr, jnp.int32) >> 1), jnp.float32)
        half = rr * 0.5
        for _ in range(3):
            w = w * (1.5 - half * w * w)
        r = rr * w
        va = plsc.load_gather(v_loc, [a])
        plsc.addupdate_scatter(acc_loc, [seg], r * va)
        return carry

    lax.fori_loop(0, NCHUNK, chunk_body, 0)

    pltpu.sync_copy(acc_loc, shared_acc.at[pl.ds(wid * N_SYS, N_SYS)])
    plsc.subcore_barrier()

    @pl.when(wid == 0)
    def _():
        pltpu.sync_copy(shared_acc, red_loc)
        pltpu.sync_copy(bvec_hbm, bvec_loc)
        tot = bvec_loc[...]
        for i in range(NTILES):
            tot = tot + red_loc[pl.ds(i * N_SYS, N_SYS)]
        tot_loc[...] = tot
        pltpu.sync_copy(tot_loc, out_hbm)


_sc_kernel = pl.kernel(
    _sc_body,
    out_type=jax.ShapeDtypeStruct((N_SYS,), jnp.float32),
    mesh=plsc.VectorSubcoreMesh(core_axis_name="c", subcore_axis_name="s",
                                num_cores=1, num_subcores=NTILES),
    compiler_params=pltpu.CompilerParams(needs_layout_passes=False),
    scratch_types=[
        pltpu.VMEM((5, APT), jnp.float32),
        pltpu.VMEM((DHALF, CHUNK), jnp.float32),
        pltpu.VMEM((DHALF,), jnp.float32),
        pltpu.VMEM((CHUNK,), jnp.float32),
        pltpu.VMEM((SPECIES_PAD,), jnp.float32),
        pltpu.VMEM((SPECIES_PAD,), jnp.float32),
        pltpu.VMEM((SPECIES_PAD,), jnp.float32),
        pltpu.VMEM((N_SYS,), jnp.float32),
        pltpu.VMEM((NTILES * N_SYS,), jnp.float32),
        pltpu.VMEM((N_SYS,), jnp.float32),
        pltpu.VMEM((N_SYS,), jnp.float32),
        pltpu.VMEM_SHARED((SPECIES_PAD,), jnp.float32),
        pltpu.VMEM_SHARED((SPECIES_PAD,), jnp.float32),
        pltpu.VMEM_SHARED((NTILES * N_SYS,), jnp.float32),
        pltpu.SemaphoreType.DMA,
    ],
)


def kernel(atomic_numbers, pos, batch, species_embed, W, b):
    posf = pos.astype(jnp.float32)
    an_f = lax.bitcast_convert_type(atomic_numbers.astype(jnp.int32), jnp.float32)
    bt_f = lax.bitcast_convert_type(batch.astype(jnp.int32), jnp.float32)
    packed = jnp.stack([posf[:, 0], posf[:, 1], posf[:, 2], an_f, bt_f])
    epad = jnp.pad(species_embed.astype(jnp.float32),
                   ((0, SPECIES_PAD - N_SPECIES), (0, 0)))
    eslab = (epad.reshape(NGROUP, CHUNK, 2, DHALF)
                 .transpose(0, 2, 3, 1)
                 .reshape(2 * NGROUP, DHALF, CHUNK))
    wf = W.reshape(-1).astype(jnp.float32)
    bvec = jnp.broadcast_to(b.astype(jnp.float32), (N_SYS,))
    y = _sc_kernel(packed, eslab, wf, bvec)
    return y.reshape(N_SYS, 1)


# speedup vs baseline: 1.0711x; 1.0159x over previous
import jax
import jax.numpy as jnp
from jax import lax
from jax.experimental import pallas as pl
from jax.experimental.pallas import tpu as pltpu
from jax.experimental.pallas import tpu_sc as plsc

N_ATOMS = 16384
N_SYS = 16
D = 512
N_SPECIES = 100

NTILES = 16
CHUNK = 16
NGROUP = 7
SPECIES_PAD = NGROUP * CHUNK
DHALF = D // 2
APT = N_ATOMS // NTILES
NCHUNK = APT // CHUNK

_MAGIC = 0x5F3759DF


def _sc_body(pk_hbm, e_hbm, w_hbm, bvec_hbm, out_hbm,
             pk_loc, e_loc, w_loc, vchunk, v0_loc, v1_loc, v_loc,
             acc_loc, red_loc, bvec_loc, tot_loc,
             shared_v0, shared_v1, shared_acc, sem):
    wid = lax.axis_index("s")

    cp_pk = pltpu.async_copy(pk_hbm.at[:, pl.ds(wid * APT, APT)], pk_loc, sem)

    @pl.when(wid < 2 * NGROUP)
    def _():
        g = wid // 2
        h = wid % 2
        pltpu.sync_copy(e_hbm.at[wid], e_loc)
        pltpu.sync_copy(w_hbm.at[pl.ds(h * DHALF, DHALF)], w_loc)
        vreg = jnp.zeros((CHUNK,), jnp.float32)
        for d0 in range(0, DHALF, CHUNK):
            wv = w_loc[pl.ds(d0, CHUNK)]
            for j in range(CHUNK):
                vreg = vreg + wv[j] * e_loc[d0 + j]
        vchunk[...] = vreg

        @pl.when(h == 0)
        def _():
            pltpu.sync_copy(vchunk, shared_v0.at[pl.ds(g * CHUNK, CHUNK)])

        @pl.when(h == 1)
        def _():
            pltpu.sync_copy(vchunk, shared_v1.at[pl.ds(g * CHUNK, CHUNK)])

    plsc.subcore_barrier()
    pltpu.sync_copy(shared_v0, v0_loc)
    pltpu.sync_copy(shared_v1, v1_loc)
    for c in range(NGROUP):
        sl = pl.ds(c * CHUNK, CHUNK)
        v_loc[sl] = v0_loc[sl] + v1_loc[sl]

    cp_pk.wait()

    acc_loc[...] = jnp.zeros((N_SYS,), jnp.float32)
    seg0_vec = plsc.bitcast(pk_loc[4, pl.ds(0, CHUNK)], jnp.int32)

    def chunk_body(c, carry):
        vacc, cur = carry
        base = c * CHUNK
        sl = pl.ds(base, CHUNK)
        x = pk_loc[0, sl]
        y = pk_loc[1, sl]
        z = pk_loc[2, sl]
        a = plsc.bitcast(pk_loc[3, sl], jnp.int32)
        seg = plsc.bitcast(pk_loc[4, sl], jnp.int32)
        rr = x * x + y * y + z * z
        w = plsc.bitcast(_MAGIC - (plsc.bitcast(rr, jnp.int32) >> 1), jnp.float32)
        half = rr * 0.5
        for _ in range(3):
            w = w * (1.5 - half * w * w)
        r = rr * w
        va = plsc.load_gather(v_loc, [a])
        val = r * va
        uniform = (seg[0] == cur) & (seg[CHUNK - 1] == cur)

        def fast():
            return vacc + val, cur

        def slow():
            plsc.addupdate_scatter(acc_loc, [jnp.full((CHUNK,), cur, jnp.int32)],
                                   vacc)
            plsc.addupdate_scatter(acc_loc, [seg], val)
            return jnp.zeros((CHUNK,), jnp.float32), seg[CHUNK - 1]

        return lax.cond(uniform, fast, slow)

    vacc, cur = lax.fori_loop(
        0, NCHUNK, chunk_body,
        (jnp.zeros((CHUNK,), jnp.float32), seg0_vec[0]))
    plsc.addupdate_scatter(acc_loc, [jnp.full((CHUNK,), cur, jnp.int32)], vacc)

    pltpu.sync_copy(acc_loc, shared_acc.at[pl.ds(wid * N_SYS, N_SYS)])
    plsc.subcore_barrier()

    @pl.when(wid == 0)
    def _():
        pltpu.sync_copy(shared_acc, red_loc)
        pltpu.sync_copy(bvec_hbm, bvec_loc)
        tot = bvec_loc[...]
        for i in range(NTILES):
            tot = tot + red_loc[pl.ds(i * N_SYS, N_SYS)]
        tot_loc[...] = tot
        pltpu.sync_copy(tot_loc, out_hbm)


_sc_kernel = pl.kernel(
    _sc_body,
    out_type=jax.ShapeDtypeStruct((N_SYS,), jnp.float32),
    mesh=plsc.VectorSubcoreMesh(core_axis_name="c", subcore_axis_name="s",
                                num_cores=1, num_subcores=NTILES),
    compiler_params=pltpu.CompilerParams(needs_layout_passes=False),
    scratch_types=[
        pltpu.VMEM((5, APT), jnp.float32),
        pltpu.VMEM((DHALF, CHUNK), jnp.float32),
        pltpu.VMEM((DHALF,), jnp.float32),
        pltpu.VMEM((CHUNK,), jnp.float32),
        pltpu.VMEM((SPECIES_PAD,), jnp.float32),
        pltpu.VMEM((SPECIES_PAD,), jnp.float32),
        pltpu.VMEM((SPECIES_PAD,), jnp.float32),
        pltpu.VMEM((N_SYS,), jnp.float32),
        pltpu.VMEM((NTILES * N_SYS,), jnp.float32),
        pltpu.VMEM((N_SYS,), jnp.float32),
        pltpu.VMEM((N_SYS,), jnp.float32),
        pltpu.VMEM_SHARED((SPECIES_PAD,), jnp.float32),
        pltpu.VMEM_SHARED((SPECIES_PAD,), jnp.float32),
        pltpu.VMEM_SHARED((NTILES * N_SYS,), jnp.float32),
        pltpu.SemaphoreType.DMA,
    ],
)


def kernel(atomic_numbers, pos, batch, species_embed, W, b):
    posf = pos.astype(jnp.float32)
    an_f = lax.bitcast_convert_type(atomic_numbers.astype(jnp.int32), jnp.float32)
    bt_f = lax.bitcast_convert_type(batch.astype(jnp.int32), jnp.float32)
    packed = jnp.stack([posf[:, 0], posf[:, 1], posf[:, 2], an_f, bt_f])
    epad = jnp.pad(species_embed.astype(jnp.float32),
                   ((0, SPECIES_PAD - N_SPECIES), (0, 0)))
    eslab = (epad.reshape(NGROUP, CHUNK, 2, DHALF)
                 .transpose(0, 2, 3, 1)
                 .reshape(2 * NGROUP, DHALF, CHUNK))
    wf = W.reshape(-1).astype(jnp.float32)
    bvec = jnp.broadcast_to(b.astype(jnp.float32), (N_SYS,))
    y = _sc_kernel(packed, eslab, wf, bvec)
    return y.reshape(N_SYS, 1)


# speedup vs baseline: 1.1132x; 1.0393x over previous
import jax
import jax.numpy as jnp
from jax import lax
from jax.experimental import pallas as pl
from jax.experimental.pallas import tpu as pltpu
from jax.experimental.pallas import tpu_sc as plsc

N_ATOMS = 16384
N_SYS = 16
D = 512
N_SPECIES = 100

NTILES = 16
CHUNK = 16
NGROUP = 7
SPECIES_PAD = NGROUP * CHUNK
DHALF = D // 2
APT = N_ATOMS // NTILES
NCHUNK = APT // CHUNK

_MAGIC = 0x5F3759DF


def _sc_body(pk_hbm, e_hbm, w_hbm, bvec_hbm, out_hbm,
             pk_loc, e_loc, w_loc, vchunk, v01_loc, v_loc,
             acc_loc, red_loc, bvec_loc, tot_loc,
             shared_v01, shared_acc, sem_pk, sem_ew, sem_b):
    wid = lax.axis_index("s")

    cp_pk = pltpu.async_copy(pk_hbm.at[:, pl.ds(wid * APT, APT)], pk_loc, sem_pk)

    @pl.when(wid == 0)
    def _():
        pltpu.async_copy(bvec_hbm, bvec_loc, sem_b)

    @pl.when(wid < 2 * NGROUP)
    def _():
        g = wid // 2
        h = wid % 2
        cp_e = pltpu.async_copy(e_hbm.at[wid], e_loc, sem_ew)
        cp_w = pltpu.async_copy(w_hbm.at[pl.ds(h * DHALF, DHALF)], w_loc, sem_ew)
        cp_e.wait()
        cp_w.wait()
        vreg = jnp.zeros((CHUNK,), jnp.float32)
        for d0 in range(0, DHALF, CHUNK):
            wv = w_loc[pl.ds(d0, CHUNK)]
            for j in range(CHUNK):
                vreg = vreg + wv[j] * e_loc[d0 + j]
        vchunk[...] = vreg
        pltpu.sync_copy(
            vchunk, shared_v01.at[pl.ds(h * SPECIES_PAD + g * CHUNK, CHUNK)])

    plsc.subcore_barrier()
    pltpu.sync_copy(shared_v01, v01_loc)
    for c in range(NGROUP):
        sl = pl.ds(c * CHUNK, CHUNK)
        v_loc[sl] = v01_loc[sl] + v01_loc[pl.ds(SPECIES_PAD + c * CHUNK, CHUNK)]

    cp_pk.wait()

    acc_loc[...] = jnp.zeros((N_SYS,), jnp.float32)
    seg0_vec = plsc.bitcast(pk_loc[4, pl.ds(0, CHUNK)], jnp.int32)

    def chunk_body(c, carry):
        vacc, cur = carry
        base = c * CHUNK
        sl = pl.ds(base, CHUNK)
        x = pk_loc[0, sl]
        y = pk_loc[1, sl]
        z = pk_loc[2, sl]
        a = plsc.bitcast(pk_loc[3, sl], jnp.int32)
        seg = plsc.bitcast(pk_loc[4, sl], jnp.int32)
        rr = x * x + y * y + z * z
        w = plsc.bitcast(_MAGIC - (plsc.bitcast(rr, jnp.int32) >> 1), jnp.float32)
        half = rr * 0.5
        for _ in range(3):
            w = w * (1.5 - half * w * w)
        r = rr * w
        va = plsc.load_gather(v_loc, [a])
        val = r * va
        uniform = (seg[0] == cur) & (seg[CHUNK - 1] == cur)

        def fast():
            return vacc + val, cur

        def slow():
            plsc.addupdate_scatter(acc_loc, [jnp.full((CHUNK,), cur, jnp.int32)],
                                   vacc)
            plsc.addupdate_scatter(acc_loc, [seg], val)
            return jnp.zeros((CHUNK,), jnp.float32), seg[CHUNK - 1]

        return lax.cond(uniform, fast, slow)

    vacc, cur = lax.fori_loop(
        0, NCHUNK, chunk_body,
        (jnp.zeros((CHUNK,), jnp.float32), seg0_vec[0]))
    plsc.addupdate_scatter(acc_loc, [jnp.full((CHUNK,), cur, jnp.int32)], vacc)

    pltpu.sync_copy(acc_loc, shared_acc.at[pl.ds(wid * N_SYS, N_SYS)])
    plsc.subcore_barrier()

    @pl.when(wid == 0)
    def _():
        pltpu.sync_copy(shared_acc, red_loc)
        pltpu.make_async_copy(bvec_hbm, bvec_loc, sem_b).wait()
        tot = bvec_loc[...]
        for i in range(NTILES):
            tot = tot + red_loc[pl.ds(i * N_SYS, N_SYS)]
        tot_loc[...] = tot
        pltpu.sync_copy(tot_loc, out_hbm)


_sc_kernel = pl.kernel(
    _sc_body,
    out_type=jax.ShapeDtypeStruct((N_SYS,), jnp.float32),
    mesh=plsc.VectorSubcoreMesh(core_axis_name="c", subcore_axis_name="s",
                                num_cores=1, num_subcores=NTILES),
    compiler_params=pltpu.CompilerParams(needs_layout_passes=False),
    scratch_types=[
        pltpu.VMEM((5, APT), jnp.float32),
        pltpu.VMEM((DHALF, CHUNK), jnp.float32),
        pltpu.VMEM((DHALF,), jnp.float32),
        pltpu.VMEM((CHUNK,), jnp.float32),
        pltpu.VMEM((2 * SPECIES_PAD,), jnp.float32),
        pltpu.VMEM((SPECIES_PAD,), jnp.float32),
        pltpu.VMEM((N_SYS,), jnp.float32),
        pltpu.VMEM((NTILES * N_SYS,), jnp.float32),
        pltpu.VMEM((N_SYS,), jnp.float32),
        pltpu.VMEM((N_SYS,), jnp.float32),
        pltpu.VMEM_SHARED((2 * SPECIES_PAD,), jnp.float32),
        pltpu.VMEM_SHARED((NTILES * N_SYS,), jnp.float32),
        pltpu.SemaphoreType.DMA,
        pltpu.SemaphoreType.DMA,
        pltpu.SemaphoreType.DMA,
    ],
)


def kernel(atomic_numbers, pos, batch, species_embed, W, b):
    posf = pos.astype(jnp.float32)
    an_f = lax.bitcast_convert_type(atomic_numbers.astype(jnp.int32), jnp.float32)
    bt_f = lax.bitcast_convert_type(batch.astype(jnp.int32), jnp.float32)
    packed = jnp.stack([posf[:, 0], posf[:, 1], posf[:, 2], an_f, bt_f])
    epad = jnp.pad(species_embed.astype(jnp.float32),
                   ((0, SPECIES_PAD - N_SPECIES), (0, 0)))
    eslab = (epad.reshape(NGROUP, CHUNK, 2, DHALF)
                 .transpose(0, 2, 3, 1)
                 .reshape(2 * NGROUP, DHALF, CHUNK))
    wf = W.reshape(-1).astype(jnp.float32)
    bvec = jnp.broadcast_to(b.astype(jnp.float32), (N_SYS,))
    y = _sc_kernel(packed, eslab, wf, bvec)
    return y.reshape(N_SYS, 1)
